# baseline (speedup 1.0000x reference)
import functools

import jax
import jax.numpy as jnp
from jax import lax
from jax.experimental import pallas as pl
from jax.experimental.pallas import tpu as pltpu
from jax.experimental.pallas import tpu_sc as plsc

_VOCAB = 1000000
_DIM = 64
_SEQ = 200
_NC = 2
_NS = 16
_NW = _NC * _NS
_L = 16
_TR = _SEQ // 8


def _mesh():
    return plsc.VectorSubcoreMesh(
        core_axis_name="c", subcore_axis_name="s",
        num_cores=_NC, num_subcores=_NS)


def _wid():
    return lax.axis_index("s") * _NC + lax.axis_index("c")


def _detile_body(textT_hbm, idx_hbm, stage_v, sem):
    wid = _wid()
    cps = [
        pltpu.async_copy(
            textT_hbm.at[pl.ds(i * 8, 8), pl.ds(wid * 128, 128)],
            stage_v.at[i], sem)
        for i in range(_TR)
    ]
    for cp in cps:
        cp.wait()
    pltpu.sync_copy(stage_v, idx_hbm.at[pl.ds(wid * _TR, _TR)])


_TCB = 512


def _tc_transpose_body(in_ref, out_ref):
    x = in_ref[...]
    y3 = x.reshape(_DIM, _TCB // 2, 2)
    z = jnp.transpose(y3, (1, 2, 0))
    out_ref[...] = z.reshape(_TCB // 2, 2 * _DIM) * (1.0 / _SEQ)


def _emb_mean_body(bpw, idx_hbm, table_hbm, out_hbm, idx_v, acc_v, sem):
    wid = _wid()
    b0 = wid * bpw

    pltpu.sync_copy(idx_hbm.at[pl.ds(wid * _TR, _TR)], idx_v)

    def zbody(r, _):
        for k in range(_DIM // _L):
            acc_v[r, pl.ds(k * _L, _L)] = jnp.zeros((_L,), jnp.float32)
        return 0

    lax.fori_loop(0, bpw, zbody, 0)

    def fire(i):
        for r in range(8):
            pltpu.async_copy(table_hbm.at[idx_v.at[i, r]], acc_v, sem,
                             add=True)

    def drain():
        for _ in range(8):
            pltpu.make_async_copy(table_hbm.at[pl.ds(0, bpw)], acc_v,
                                  sem).wait()

    fire(0)

    def body(i, _):
        fire(i)
        drain()
        return 0

    lax.fori_loop(1, _TR, body, 0)
    drain()
    pltpu.sync_copy(acc_v, out_hbm.at[pl.ds(b0, bpw)])


@functools.partial(jax.jit, static_argnames=("batch",))
def _emb_mean(textT, tableT, batch):
    bpw = batch // _NW
    idx = pl.kernel(
        _detile_body,
        out_type=jax.ShapeDtypeStruct((_NW * _TR, 8, 128), jnp.int32),
        mesh=_mesh(),
        compiler_params=pltpu.CompilerParams(use_tc_tiling_on_sc=True),
        scratch_types=[
            pltpu.VMEM((_TR, 8, 128), jnp.int32),
            pltpu.SemaphoreType.DMA,
        ],
    )(textT)
    comp = pl.pallas_call(
        _tc_transpose_body,
        grid=((_VOCAB // 2 + _TCB // 2 - 1) // (_TCB // 2),),
        in_specs=[pl.BlockSpec((_DIM, _TCB), lambda i: (0, i))],
        out_specs=pl.BlockSpec((_TCB // 2, 2 * _DIM), lambda i: (i, 0)),
        out_shape=jax.ShapeDtypeStruct((_VOCAB // 2, 2 * _DIM), jnp.float32),
    )(tableT)
    rm = jnp.reshape(comp, (_VOCAB, _DIM))
    return pl.kernel(
        functools.partial(_emb_mean_body, bpw),
        out_type=jax.ShapeDtypeStruct((batch, _DIM), jnp.float32),
        mesh=_mesh(),
        compiler_params=pltpu.CompilerParams(use_tc_tiling_on_sc=False),
        scratch_types=[
            pltpu.VMEM((_TR, 8, 128), jnp.int32),
            pltpu.VMEM((bpw, _DIM), jnp.float32),
            pltpu.SemaphoreType.DMA,
        ],
    )(idx, rm)


def kernel(text, table):
    batch = text.shape[0]
    textT = jnp.swapaxes(text.astype(jnp.int32), 0, 1)
    tableT = jnp.swapaxes(table, 0, 1)
    return _emb_mean(textT, tableT, batch)


# speedup vs baseline: 13.2569x; 13.2569x over previous
import functools

import jax
import jax.numpy as jnp
from jax import lax
from jax.experimental import pallas as pl
from jax.experimental.pallas import tpu as pltpu
from jax.experimental.pallas import tpu_sc as plsc

_VOCAB = 1000000
_DIM = 64
_SEQ = 200
_NC = 2
_NS = 16
_NW = _NC * _NS
_L = 16
_TR = _SEQ // 8


def _mesh():
    return plsc.VectorSubcoreMesh(
        core_axis_name="c", subcore_axis_name="s",
        num_cores=_NC, num_subcores=_NS)


def _wid():
    return lax.axis_index("s") * _NC + lax.axis_index("c")


def _detile_body(textT_hbm, idx_hbm, stage_v, sem):
    wid = _wid()
    cps = [
        pltpu.async_copy(
            textT_hbm.at[pl.ds(i * 8, 8), pl.ds(wid * 128, 128)],
            stage_v.at[i], sem)
        for i in range(_TR)
    ]
    for cp in cps:
        cp.wait()
    pltpu.sync_copy(stage_v, idx_hbm.at[pl.ds(wid * _TR, _TR)])


def _emb_mean_body(bpw, idx_hbm, table_hbm, out_hbm, idx_v, acc_v, sem):
    wid = _wid()
    b0 = wid * bpw

    pltpu.sync_copy(idx_hbm.at[pl.ds(wid * _TR, _TR)], idx_v)

    def zbody(r, _):
        for k in range(_DIM // _L):
            acc_v[r, pl.ds(k * _L, _L)] = jnp.zeros((_L,), jnp.float32)
        return 0

    lax.fori_loop(0, bpw, zbody, 0)

    def fire(i):
        for r in range(8):
            pltpu.async_copy(table_hbm.at[idx_v.at[i, r]], acc_v, sem,
                             add=True)

    def drain():
        for _ in range(8):
            pltpu.make_async_copy(table_hbm.at[pl.ds(0, bpw)], acc_v,
                                  sem).wait()

    fire(0)

    def body(i, _):
        fire(i)
        drain()
        return 0

    lax.fori_loop(1, _TR, body, 0)
    drain()

    scale = jnp.float32(1.0 / _SEQ)

    def sbody(r, _):
        for k in range(_DIM // _L):
            acc_v[r, pl.ds(k * _L, _L)] = acc_v[r, pl.ds(k * _L, _L)] * scale
        return 0

    lax.fori_loop(0, bpw, sbody, 0)
    pltpu.sync_copy(acc_v, out_hbm.at[pl.ds(b0, bpw)])


@functools.partial(jax.jit, static_argnames=("batch",))
def _emb_mean(textT, table, batch):
    bpw = batch // _NW
    idx = pl.kernel(
        _detile_body,
        out_type=jax.ShapeDtypeStruct((_NW * _TR, 8, 128), jnp.int32),
        mesh=_mesh(),
        compiler_params=pltpu.CompilerParams(use_tc_tiling_on_sc=True),
        scratch_types=[
            pltpu.VMEM((_TR, 8, 128), jnp.int32),
            pltpu.SemaphoreType.DMA,
        ],
    )(textT)
    return pl.kernel(
        functools.partial(_emb_mean_body, bpw),
        out_type=jax.ShapeDtypeStruct((batch, _DIM), jnp.float32),
        mesh=_mesh(),
        compiler_params=pltpu.CompilerParams(use_tc_tiling_on_sc=False),
        scratch_types=[
            pltpu.VMEM((_TR, 8, 128), jnp.int32),
            pltpu.VMEM((bpw, _DIM), jnp.float32),
            pltpu.SemaphoreType.DMA,
        ],
    )(idx, table)


def kernel(text, table):
    batch = text.shape[0]
    textT = jnp.swapaxes(text.astype(jnp.int32), 0, 1)
    return _emb_mean(textT, table, batch)
